# baseline (speedup 1.0000x reference)
import functools

import jax
import jax.numpy as jnp
from jax import lax
from jax.experimental import pallas as pl
from jax.experimental.pallas import tpu as pltpu
from jax.experimental.pallas import tpu_sc as plsc

_LANES = 16


def _counts_sc_kernel(indices, V):
    B, L = indices.shape

    info = plsc.get_sparse_core_info()
    NC, NS = info.num_cores, info.num_subcores
    NW = NC * NS

    rows_per_w = B // NW
    BR = 16
    n_blocks = rows_per_w // BR
    n_lc = -(-L // _LANES)
    tail = L % _LANES

    mesh = plsc.VectorSubcoreMesh(core_axis_name="c", subcore_axis_name="s")

    def body(idx_hbm, out_hbm, idx_v, cnt_v, isem, osem):
        wid = lax.axis_index("s") * NC + lax.axis_index("c")
        base_row = wid * rows_per_w

        row_iota = lax.iota(jnp.int32, _LANES)
        ones = jnp.ones((_LANES,), jnp.float32)

        n_zc = -(-V // _LANES)
        zeros16 = jnp.zeros((_LANES,), jnp.float32)

        def zero_row(r, c):
            rvec = jnp.full((_LANES,), r, jnp.int32)

            def zero_col(z, c2):
                start = jnp.minimum(z * _LANES, V - _LANES)
                plsc.store_scatter(cnt_v, [rvec, start + row_iota], zeros16)
                return c2

            lax.fori_loop(0, n_zc, zero_col, 0)
            return c

        lax.fori_loop(0, 2 * BR, zero_row, 0)

        def idx_dma(t, slot):
            return pltpu.make_async_copy(
                idx_hbm.at[pl.ds(base_row + t * BR, BR), :],
                idx_v.at[pl.ds(slot * BR, BR), :],
                isem,
            )

        def cnt_dma(t, slot):
            return pltpu.make_async_copy(
                cnt_v.at[pl.ds(slot * BR, BR), :],
                out_hbm.at[pl.ds(base_row + t * BR, BR), :],
                osem,
            )

        def scan_rows(slot, kbase, val):
            def k_body(k, c):
                kvec = jnp.full((_LANES,), kbase + k, jnp.int32)

                def lc_body(lc, c2):
                    start = lc * _LANES - jnp.where(
                        lc == n_lc - 1, _LANES - tail if tail else 0, 0
                    )
                    vidx = plsc.load_gather(
                        idx_v, [jnp.full((_LANES,), slot * BR + k, jnp.int32),
                                start + row_iota]
                    )
                    thr = jnp.where(lc == n_lc - 1, _LANES - tail if tail else 0, 0)
                    mask = row_iota >= thr
                    plsc.addupdate_scatter(cnt_v, [kvec, vidx], val, mask=mask)
                    return c2

                lax.fori_loop(0, n_lc, lc_body, 0)
                return c

            lax.fori_loop(0, BR, k_body, 0)

        idx_dma(0, 0).start()
        idx_dma(0, 0).wait()
        idx_dma(1, 1).start()
        scan_rows(0, 0, ones)
        cnt_dma(0, 0).start()

        def block_body(t, c):
            buf = lax.rem(t, 2)
            prev = 1 - buf
            idx_dma(t, buf).wait()
            cnt_dma(t - 1, prev).wait()
            scan_rows(prev, prev * BR, -ones)

            @pl.when(t + 1 < n_blocks)
            def _():
                idx_dma(t + 1, prev).start()

            scan_rows(buf, buf * BR, ones)
            cnt_dma(t, buf).start()
            return c

        lax.fori_loop(1, n_blocks, block_body, 0)
        cnt_dma(n_blocks - 1, lax.rem(n_blocks - 1, 2)).wait()

    return pl.kernel(
        body,
        out_type=jax.ShapeDtypeStruct((B, V), jnp.float32),
        mesh=mesh,
        compiler_params=pltpu.CompilerParams(needs_layout_passes=False),
        scratch_types=[
            pltpu.VMEM((2 * BR, L), jnp.int32),
            pltpu.VMEM((2 * BR, V), jnp.float32),
            pltpu.SemaphoreType.DMA,
            pltpu.SemaphoreType.DMA,
        ],
    )(indices)


def _matmul_mlp_body(c_ref, t_ref, w1_ref, b1_ref, w2_ref, b2_ref,
                     w3_ref, b3_ref, o_ref, *, inv_l):
    x = jnp.dot(
        c_ref[...], t_ref[...], preferred_element_type=jnp.float32
    ) * inv_l
    h1 = jnp.maximum(
        jnp.dot(x, w1_ref[...], preferred_element_type=jnp.float32) + b1_ref[...], 0.0
    )
    h2 = jnp.maximum(
        jnp.dot(h1, w2_ref[...], preferred_element_type=jnp.float32) + b2_ref[...], 0.0
    )
    o_ref[...] = jax.nn.sigmoid(
        jnp.dot(h2, w3_ref[...], preferred_element_type=jnp.float32) + b3_ref[...]
    )


def _matmul_mlp_tc(counts, emb_table, L, W1, b1, W2, b2, W3, b3):
    B, V = counts.shape
    D = emb_table.shape[1]
    H1 = W1.shape[1]
    H2 = W2.shape[1]
    TB = 2048
    grid = (B // TB,)
    b1r, b2r, b3r = b1.reshape(1, H1), b2.reshape(1, H2), b3.reshape(1, 1)
    fixed = lambda i: (0, 0)
    return pl.pallas_call(
        functools.partial(_matmul_mlp_body, inv_l=1.0 / L),
        grid=grid,
        in_specs=[
            pl.BlockSpec((TB, V), lambda i: (i, 0)),
            pl.BlockSpec((V, D), fixed),
            pl.BlockSpec((D, H1), fixed),
            pl.BlockSpec((1, H1), fixed),
            pl.BlockSpec((H1, H2), fixed),
            pl.BlockSpec((1, H2), fixed),
            pl.BlockSpec((H2, 1), fixed),
            pl.BlockSpec((1, 1), fixed),
        ],
        out_specs=pl.BlockSpec((TB, 1), lambda i: (i, 0)),
        out_shape=jax.ShapeDtypeStruct((B, 1), jnp.float32),
    )(counts, emb_table, W1, b1r, W2, b2r, W3, b3r)


def kernel(indices, emb_table, W1, b1, W2, b2, W3, b3):
    V = emb_table.shape[0]
    L = indices.shape[1]
    counts = _counts_sc_kernel(indices, V)
    return _matmul_mlp_tc(counts, emb_table, L, W1, b1, W2, b2, W3, b3)


# speedup vs baseline: 1.0555x; 1.0555x over previous
import functools

import jax
import jax.numpy as jnp
from jax import lax
from jax.experimental import pallas as pl
from jax.experimental.pallas import tpu as pltpu
from jax.experimental.pallas import tpu_sc as plsc

_LANES = 16


def _counts_sc_kernel(indices, V):
    B, L = indices.shape

    info = plsc.get_sparse_core_info()
    NC, NS = info.num_cores, info.num_subcores
    NW = NC * NS

    rows_per_w = B // NW
    BR = 16
    n_blocks = rows_per_w // BR
    n_lc = -(-L // _LANES)
    tail = L % _LANES

    mesh = plsc.VectorSubcoreMesh(core_axis_name="c", subcore_axis_name="s")

    def body(idx_hbm, out_hbm, idx_v, cnt_v, isem, osem):
        wid = lax.axis_index("s") * NC + lax.axis_index("c")
        base_row = wid * rows_per_w

        row_iota = lax.iota(jnp.int32, _LANES)
        ones = jnp.ones((_LANES,), jnp.float32)

        n_zc = -(-V // _LANES)
        zeros16 = jnp.zeros((_LANES,), jnp.float32)
        zcols = [min(z * _LANES, V - _LANES) + row_iota for z in range(n_zc)]

        def zero_row(r, c):
            rvec = jnp.full((_LANES,), r, jnp.int32)
            for z in range(n_zc):
                plsc.store_scatter(cnt_v, [rvec, zcols[z]], zeros16)
            return c

        lax.fori_loop(0, 2 * BR, zero_row, 0)

        def idx_dma(t, slot):
            return pltpu.make_async_copy(
                idx_hbm.at[pl.ds(base_row + t * BR, BR), :],
                idx_v.at[pl.ds(slot * BR, BR), :],
                isem,
            )

        def cnt_dma(t, slot):
            return pltpu.make_async_copy(
                cnt_v.at[pl.ds(slot * BR, BR), :],
                out_hbm.at[pl.ds(base_row + t * BR, BR), :],
                osem,
            )

        lstarts = [
            min(lc * _LANES, L - _LANES) + row_iota for lc in range(n_lc)
        ]
        tail_mask = row_iota >= (_LANES - tail if tail else 0)

        def scan_rows(slot, kbase, val):
            def k_body(k, c):
                kvec = jnp.full((_LANES,), kbase + k, jnp.int32)
                rowvec = jnp.full((_LANES,), slot * BR + k, jnp.int32)
                for lc in range(n_lc):
                    vidx = plsc.load_gather(idx_v, [rowvec, lstarts[lc]])
                    mask = tail_mask if lc == n_lc - 1 else None
                    plsc.addupdate_scatter(cnt_v, [kvec, vidx], val, mask=mask)
                return c

            lax.fori_loop(0, BR, k_body, 0)

        idx_dma(0, 0).start()
        idx_dma(0, 0).wait()
        idx_dma(1, 1).start()
        scan_rows(0, 0, ones)
        cnt_dma(0, 0).start()

        def block_body(t, c):
            buf = lax.rem(t, 2)
            prev = 1 - buf
            idx_dma(t, buf).wait()
            cnt_dma(t - 1, prev).wait()
            scan_rows(prev, prev * BR, -ones)

            @pl.when(t + 1 < n_blocks)
            def _():
                idx_dma(t + 1, prev).start()

            scan_rows(buf, buf * BR, ones)
            cnt_dma(t, buf).start()
            return c

        lax.fori_loop(1, n_blocks, block_body, 0)
        cnt_dma(n_blocks - 1, lax.rem(n_blocks - 1, 2)).wait()

    return pl.kernel(
        body,
        out_type=jax.ShapeDtypeStruct((B, V), jnp.float32),
        mesh=mesh,
        compiler_params=pltpu.CompilerParams(needs_layout_passes=False),
        scratch_types=[
            pltpu.VMEM((2 * BR, L), jnp.int32),
            pltpu.VMEM((2 * BR, V), jnp.float32),
            pltpu.SemaphoreType.DMA,
            pltpu.SemaphoreType.DMA,
        ],
    )(indices)


def _matmul_mlp_body(c_ref, t_ref, w1_ref, b1_ref, w2_ref, b2_ref,
                     w3_ref, b3_ref, o_ref, *, inv_l):
    x = jnp.dot(
        c_ref[...], t_ref[...], preferred_element_type=jnp.float32
    ) * inv_l
    h1 = jnp.maximum(
        jnp.dot(x, w1_ref[...], preferred_element_type=jnp.float32) + b1_ref[...], 0.0
    )
    h2 = jnp.maximum(
        jnp.dot(h1, w2_ref[...], preferred_element_type=jnp.float32) + b2_ref[...], 0.0
    )
    o_ref[...] = jax.nn.sigmoid(
        jnp.dot(h2, w3_ref[...], preferred_element_type=jnp.float32) + b3_ref[...]
    )


def _matmul_mlp_tc(counts, emb_table, L, W1, b1, W2, b2, W3, b3):
    B, V = counts.shape
    D = emb_table.shape[1]
    H1 = W1.shape[1]
    H2 = W2.shape[1]
    TB = 2048
    grid = (B // TB,)
    b1r, b2r, b3r = b1.reshape(1, H1), b2.reshape(1, H2), b3.reshape(1, 1)
    fixed = lambda i: (0, 0)
    return pl.pallas_call(
        functools.partial(_matmul_mlp_body, inv_l=1.0 / L),
        grid=grid,
        in_specs=[
            pl.BlockSpec((TB, V), lambda i: (i, 0)),
            pl.BlockSpec((V, D), fixed),
            pl.BlockSpec((D, H1), fixed),
            pl.BlockSpec((1, H1), fixed),
            pl.BlockSpec((H1, H2), fixed),
            pl.BlockSpec((1, H2), fixed),
            pl.BlockSpec((H2, 1), fixed),
            pl.BlockSpec((1, 1), fixed),
        ],
        out_specs=pl.BlockSpec((TB, 1), lambda i: (i, 0)),
        out_shape=jax.ShapeDtypeStruct((B, 1), jnp.float32),
    )(counts, emb_table, W1, b1r, W2, b2r, W3, b3r)


def kernel(indices, emb_table, W1, b1, W2, b2, W3, b3):
    V = emb_table.shape[0]
    L = indices.shape[1]
    counts = _counts_sc_kernel(indices, V)
    return _matmul_mlp_tc(counts, emb_table, L, W1, b1, W2, b2, W3, b3)


# speedup vs baseline: 1.3259x; 1.2562x over previous
import functools

import jax
import jax.numpy as jnp
from jax import lax
from jax.experimental import pallas as pl
from jax.experimental.pallas import tpu as pltpu
from jax.experimental.pallas import tpu_sc as plsc

_LANES = 16


def _pooled_sc_kernel(indices, emb_table):
    B, L = indices.shape
    V, D = emb_table.shape

    info = plsc.get_sparse_core_info()
    NC, NS = info.num_cores, info.num_subcores
    NW = NC * NS

    rows_per_w = B // NW
    C = 64
    n_groups = rows_per_w // C
    n_sub = C // _LANES

    mesh = plsc.VectorSubcoreMesh(core_axis_name="c", subcore_axis_name="s")

    def body(idx_hbm, table_hbm, out_hbm, table_v, idx_v, out_v,
             isem0, isem1, osem0, osem1):
        wid = lax.axis_index("s") * NC + lax.axis_index("c")
        base_row = wid * rows_per_w
        isems = (isem0, isem1)
        osems = (osem0, osem1)

        idx_cp = [None] * n_groups
        idx_cp[0] = pltpu.async_copy(
            idx_hbm.at[pl.ds(base_row, C), :], idx_v.at[0], isems[0]
        )
        pltpu.sync_copy(table_hbm, table_v)

        row_iota = lax.iota(jnp.int32, _LANES)
        zeros = tuple(jnp.zeros((_LANES,), jnp.float32) for _ in range(D))
        inv_l = jnp.float32(1.0 / L)

        out_cp = [None] * n_groups
        for g in range(n_groups):
            buf = g % 2
            row0 = base_row + g * C
            if g + 1 < n_groups:
                idx_cp[g + 1] = pltpu.async_copy(
                    idx_hbm.at[pl.ds(row0 + C, C), :],
                    idx_v.at[1 - buf],
                    isems[1 - buf],
                )
            idx_cp[g].wait()
            if g >= 2:
                out_cp[g - 2].wait()

            def sub_body(s, carry2):
                rvec = s * _LANES + row_iota

                def l_body(l, accs):
                    lvec = jnp.full((_LANES,), l, jnp.int32)
                    vidx = plsc.load_gather(idx_v.at[buf], [rvec, lvec])
                    tbase = vidx * D
                    return tuple(
                        accs[d] + plsc.load_gather(table_v, [tbase + d])
                        for d in range(D)
                    )

                accs = lax.fori_loop(0, L, l_body, zeros, unroll=3)
                for d in range(D):
                    dvec = jnp.full((_LANES,), d, jnp.int32)
                    plsc.store_scatter(out_v.at[buf], [rvec, dvec], accs[d] * inv_l)
                return carry2

            lax.fori_loop(0, n_sub, sub_body, 0)
            out_cp[g] = pltpu.async_copy(
                out_v.at[buf], out_hbm.at[pl.ds(row0, C), :], osems[buf]
            )
        for g in range(max(0, n_groups - 2), n_groups):
            out_cp[g].wait()

    return pl.kernel(
        body,
        out_type=jax.ShapeDtypeStruct((B, D), jnp.float32),
        mesh=mesh,
        compiler_params=pltpu.CompilerParams(needs_layout_passes=False),
        scratch_types=[
            pltpu.VMEM((V * D,), jnp.float32),
            pltpu.VMEM((2, C, L), jnp.int32),
            pltpu.VMEM((2, C, D), jnp.float32),
            pltpu.SemaphoreType.DMA,
            pltpu.SemaphoreType.DMA,
            pltpu.SemaphoreType.DMA,
            pltpu.SemaphoreType.DMA,
        ],
    )(indices, emb_table.reshape(V * D))


def _mlp_body(x_ref, w1_ref, b1_ref, w2_ref, b2_ref, w3_ref, b3_ref, o_ref):
    x = x_ref[...]
    h1 = jnp.maximum(
        jnp.dot(x, w1_ref[...], preferred_element_type=jnp.float32) + b1_ref[...], 0.0
    )
    h2 = jnp.maximum(
        jnp.dot(h1, w2_ref[...], preferred_element_type=jnp.float32) + b2_ref[...], 0.0
    )
    o_ref[...] = jax.nn.sigmoid(
        jnp.dot(h2, w3_ref[...], preferred_element_type=jnp.float32) + b3_ref[...]
    )


def _mlp_tc(pooled, W1, b1, W2, b2, W3, b3):
    B, D = pooled.shape
    H1 = W1.shape[1]
    H2 = W2.shape[1]
    TB = 2048
    grid = (B // TB,)
    b1r, b2r, b3r = b1.reshape(1, H1), b2.reshape(1, H2), b3.reshape(1, 1)
    fixed = lambda i: (0, 0)
    return pl.pallas_call(
        _mlp_body,
        grid=grid,
        in_specs=[
            pl.BlockSpec((TB, D), lambda i: (i, 0)),
            pl.BlockSpec((D, H1), fixed),
            pl.BlockSpec((1, H1), fixed),
            pl.BlockSpec((H1, H2), fixed),
            pl.BlockSpec((1, H2), fixed),
            pl.BlockSpec((H2, 1), fixed),
            pl.BlockSpec((1, 1), fixed),
        ],
        out_specs=pl.BlockSpec((TB, 1), lambda i: (i, 0)),
        out_shape=jax.ShapeDtypeStruct((B, 1), jnp.float32),
    )(pooled, W1, b1r, W2, b2r, W3, b3r)


def kernel(indices, emb_table, W1, b1, W2, b2, W3, b3):
    pooled = _pooled_sc_kernel(indices, emb_table)
    return _mlp_tc(pooled, W1, b1, W2, b2, W3, b3)


# speedup vs baseline: 1.3847x; 1.0443x over previous
import functools

import jax
import jax.numpy as jnp
from jax import lax
from jax.experimental import pallas as pl
from jax.experimental.pallas import tpu as pltpu
from jax.experimental.pallas import tpu_sc as plsc

_LANES = 16


def _pooled_sc_kernel(indices, emb_table):
    B, L = indices.shape
    V, D = emb_table.shape

    info = plsc.get_sparse_core_info()
    NC, NS = info.num_cores, info.num_subcores
    NW = NC * NS

    rows_per_w = B // NW
    C = 64
    n_groups = rows_per_w // C
    n_sub = C // _LANES

    mesh = plsc.VectorSubcoreMesh(core_axis_name="c", subcore_axis_name="s")

    def body(idx_hbm, table_hbm, out_hbm, table_v, idx_v, out_v,
             isem0, isem1, osem0, osem1):
        wid = lax.axis_index("s") * NC + lax.axis_index("c")
        base_row = wid * rows_per_w
        isems = (isem0, isem1)
        osems = (osem0, osem1)

        idx_cp = [None] * n_groups
        idx_cp[0] = pltpu.async_copy(
            idx_hbm.at[pl.ds(base_row, C), :], idx_v.at[0], isems[0]
        )
        pltpu.sync_copy(table_hbm, table_v)

        row_iota = lax.iota(jnp.int32, _LANES)
        zeros = tuple(jnp.zeros((_LANES,), jnp.float32) for _ in range(D))
        inv_l = jnp.float32(1.0 / L)

        out_cp = [None] * n_groups
        for g in range(n_groups):
            buf = g % 2
            row0 = base_row + g * C
            if g + 1 < n_groups:
                idx_cp[g + 1] = pltpu.async_copy(
                    idx_hbm.at[pl.ds(row0 + C, C), :],
                    idx_v.at[1 - buf],
                    isems[1 - buf],
                )
            idx_cp[g].wait()
            if g >= 2:
                out_cp[g - 2].wait()

            def sub_body(s, carry2):
                rvec = s * _LANES + row_iota

                def accum(vidx, accs):
                    tbase = vidx * D
                    return tuple(
                        accs[d] + plsc.load_gather(table_v, [tbase + d])
                        for d in range(D)
                    )

                def l_body(l, carry):
                    vidx, accs = carry
                    nvec = jnp.full((_LANES,), l + 1, jnp.int32)
                    vidx_next = plsc.load_gather(idx_v.at[buf], [rvec, nvec])
                    return vidx_next, accum(vidx, accs)

                vidx0 = plsc.load_gather(
                    idx_v.at[buf], [rvec, jnp.zeros((_LANES,), jnp.int32)]
                )
                vlast, accs = lax.fori_loop(
                    0, L - 1, l_body, (vidx0, zeros), unroll=2
                )
                accs = accum(vlast, accs)
                for d in range(D):
                    dvec = jnp.full((_LANES,), d, jnp.int32)
                    plsc.store_scatter(out_v.at[buf], [rvec, dvec], accs[d] * inv_l)
                return carry2

            lax.fori_loop(0, n_sub, sub_body, 0)
            out_cp[g] = pltpu.async_copy(
                out_v.at[buf], out_hbm.at[pl.ds(row0, C), :], osems[buf]
            )
        for g in range(max(0, n_groups - 2), n_groups):
            out_cp[g].wait()

    return pl.kernel(
        body,
        out_type=jax.ShapeDtypeStruct((B, D), jnp.float32),
        mesh=mesh,
        compiler_params=pltpu.CompilerParams(needs_layout_passes=False),
        scratch_types=[
            pltpu.VMEM((V * D,), jnp.float32),
            pltpu.VMEM((2, C, L), jnp.int32),
            pltpu.VMEM((2, C, D), jnp.float32),
            pltpu.SemaphoreType.DMA,
            pltpu.SemaphoreType.DMA,
            pltpu.SemaphoreType.DMA,
            pltpu.SemaphoreType.DMA,
        ],
    )(indices, emb_table.reshape(V * D))


def _mlp_body(x_ref, w1_ref, b1_ref, w2_ref, b2_ref, w3_ref, b3_ref, o_ref):
    x = x_ref[...]
    h1 = jnp.maximum(
        jnp.dot(x, w1_ref[...], preferred_element_type=jnp.float32) + b1_ref[...], 0.0
    )
    h2 = jnp.maximum(
        jnp.dot(h1, w2_ref[...], preferred_element_type=jnp.float32) + b2_ref[...], 0.0
    )
    o_ref[...] = jax.nn.sigmoid(
        jnp.dot(h2, w3_ref[...], preferred_element_type=jnp.float32) + b3_ref[...]
    )


def _mlp_tc(pooled, W1, b1, W2, b2, W3, b3):
    B, D = pooled.shape
    H1 = W1.shape[1]
    H2 = W2.shape[1]
    TB = 2048
    grid = (B // TB,)
    b1r, b2r, b3r = b1.reshape(1, H1), b2.reshape(1, H2), b3.reshape(1, 1)
    fixed = lambda i: (0, 0)
    return pl.pallas_call(
        _mlp_body,
        grid=grid,
        in_specs=[
            pl.BlockSpec((TB, D), lambda i: (i, 0)),
            pl.BlockSpec((D, H1), fixed),
            pl.BlockSpec((1, H1), fixed),
            pl.BlockSpec((H1, H2), fixed),
            pl.BlockSpec((1, H2), fixed),
            pl.BlockSpec((H2, 1), fixed),
            pl.BlockSpec((1, 1), fixed),
        ],
        out_specs=pl.BlockSpec((TB, 1), lambda i: (i, 0)),
        out_shape=jax.ShapeDtypeStruct((B, 1), jnp.float32),
    )(pooled, W1, b1r, W2, b2r, W3, b3r)


def kernel(indices, emb_table, W1, b1, W2, b2, W3, b3):
    pooled = _pooled_sc_kernel(indices, emb_table)
    return _mlp_tc(pooled, W1, b1, W2, b2, W3, b3)


# speedup vs baseline: 1.4404x; 1.0402x over previous
import functools

import jax
import jax.numpy as jnp
from jax import lax
from jax.experimental import pallas as pl
from jax.experimental.pallas import tpu as pltpu
from jax.experimental.pallas import tpu_sc as plsc

_LANES = 16


def _pooled_sc_kernel(indices, emb_table):
    B, L = indices.shape
    V, D = emb_table.shape

    info = plsc.get_sparse_core_info()
    NC, NS = info.num_cores, info.num_subcores
    NW = NC * NS

    rows_per_w = B // NW
    C = 32
    n_groups = rows_per_w // C
    n_sub = C // _LANES

    mesh = plsc.VectorSubcoreMesh(core_axis_name="c", subcore_axis_name="s")

    def body(idx_hbm, table_hbm, out_hbm, table_v, idx_v, out_v,
             isem0, isem1, osem0, osem1):
        wid = lax.axis_index("s") * NC + lax.axis_index("c")
        base_row = wid * rows_per_w
        isems = (isem0, isem1)
        osems = (osem0, osem1)

        idx_cp = [None] * n_groups
        idx_cp[0] = pltpu.async_copy(
            idx_hbm.at[pl.ds(base_row, C), :], idx_v.at[0], isems[0]
        )
        pltpu.sync_copy(table_hbm, table_v)

        row_iota = lax.iota(jnp.int32, _LANES)
        zeros = tuple(jnp.zeros((_LANES,), jnp.float32) for _ in range(D))
        inv_l = jnp.float32(1.0 / L)

        out_cp = [None] * n_groups
        for g in range(n_groups):
            buf = g % 2
            row0 = base_row + g * C
            if g + 1 < n_groups:
                idx_cp[g + 1] = pltpu.async_copy(
                    idx_hbm.at[pl.ds(row0 + C, C), :],
                    idx_v.at[1 - buf],
                    isems[1 - buf],
                )
            idx_cp[g].wait()
            if g >= 2:
                out_cp[g - 2].wait()

            def sub_body(s, carry2):
                rvec = s * _LANES + row_iota

                def l_body(l, accs):
                    lvec = jnp.full((_LANES,), l, jnp.int32)
                    vidx = plsc.load_gather(idx_v.at[buf], [rvec, lvec])
                    tbase = vidx * D
                    return tuple(
                        accs[d] + plsc.load_gather(table_v, [tbase + d])
                        for d in range(D)
                    )

                accs = lax.fori_loop(0, L, l_body, zeros, unroll=2)
                for d in range(D):
                    dvec = jnp.full((_LANES,), d, jnp.int32)
                    plsc.store_scatter(out_v.at[buf], [rvec, dvec], accs[d] * inv_l)
                return carry2

            lax.fori_loop(0, n_sub, sub_body, 0)
            out_cp[g] = pltpu.async_copy(
                out_v.at[buf], out_hbm.at[pl.ds(row0, C), :], osems[buf]
            )
        for g in range(max(0, n_groups - 2), n_groups):
            out_cp[g].wait()

    return pl.kernel(
        body,
        out_type=jax.ShapeDtypeStruct((B, D), jnp.float32),
        mesh=mesh,
        compiler_params=pltpu.CompilerParams(needs_layout_passes=False),
        scratch_types=[
            pltpu.VMEM((V * D,), jnp.float32),
            pltpu.VMEM((2, C, L), jnp.int32),
            pltpu.VMEM((2, C, D), jnp.float32),
            pltpu.SemaphoreType.DMA,
            pltpu.SemaphoreType.DMA,
            pltpu.SemaphoreType.DMA,
            pltpu.SemaphoreType.DMA,
        ],
    )(indices, emb_table.reshape(V * D))


def _mlp_body(x_ref, w1_ref, b1_ref, w2_ref, b2_ref, w3_ref, b3_ref, o_ref):
    x = x_ref[...]
    h1 = jnp.maximum(
        jnp.dot(x, w1_ref[...], preferred_element_type=jnp.float32) + b1_ref[...], 0.0
    )
    h2 = jnp.maximum(
        jnp.dot(h1, w2_ref[...], preferred_element_type=jnp.float32) + b2_ref[...], 0.0
    )
    o_ref[...] = jax.nn.sigmoid(
        jnp.dot(h2, w3_ref[...], preferred_element_type=jnp.float32) + b3_ref[...]
    )


def _mlp_tc(pooled, W1, b1, W2, b2, W3, b3):
    B, D = pooled.shape
    H1 = W1.shape[1]
    H2 = W2.shape[1]
    TB = 2048
    grid = (B // TB,)
    b1r, b2r, b3r = b1.reshape(1, H1), b2.reshape(1, H2), b3.reshape(1, 1)
    fixed = lambda i: (0, 0)
    return pl.pallas_call(
        _mlp_body,
        grid=grid,
        in_specs=[
            pl.BlockSpec((TB, D), lambda i: (i, 0)),
            pl.BlockSpec((D, H1), fixed),
            pl.BlockSpec((1, H1), fixed),
            pl.BlockSpec((H1, H2), fixed),
            pl.BlockSpec((1, H2), fixed),
            pl.BlockSpec((H2, 1), fixed),
            pl.BlockSpec((1, 1), fixed),
        ],
        out_specs=pl.BlockSpec((TB, 1), lambda i: (i, 0)),
        out_shape=jax.ShapeDtypeStruct((B, 1), jnp.float32),
    )(pooled, W1, b1r, W2, b2r, W3, b3r)


def kernel(indices, emb_table, W1, b1, W2, b2, W3, b3):
    pooled = _pooled_sc_kernel(indices, emb_table)
    return _mlp_tc(pooled, W1, b1, W2, b2, W3, b3)


# speedup vs baseline: 1.4446x; 1.0029x over previous
import functools

import jax
import jax.numpy as jnp
from jax import lax
from jax.experimental import pallas as pl
from jax.experimental.pallas import tpu as pltpu
from jax.experimental.pallas import tpu_sc as plsc

_LANES = 16


def _pooled_sc_kernel(indices, emb_table):
    B, L = indices.shape
    V, D = emb_table.shape

    info = plsc.get_sparse_core_info()
    NC, NS = info.num_cores, info.num_subcores
    NW = NC * NS

    rows_per_w = B // NW
    C = 64
    n_groups = rows_per_w // C
    n_sub = C // _LANES

    mesh = plsc.VectorSubcoreMesh(core_axis_name="c", subcore_axis_name="s")

    def body(idx_hbm, table_hbm, out_hbm, table_v, idx_v, out_v,
             isem0, isem1, osem0, osem1):
        wid = lax.axis_index("s") * NC + lax.axis_index("c")
        base_row = wid * rows_per_w
        isems = (isem0, isem1)
        osems = (osem0, osem1)

        idx_cp = [None] * n_groups
        idx_cp[0] = pltpu.async_copy(
            idx_hbm.at[pl.ds(base_row, C), :], idx_v.at[0], isems[0]
        )
        pltpu.sync_copy(table_hbm, table_v)

        row_iota = lax.iota(jnp.int32, _LANES)
        zeros = tuple(jnp.zeros((_LANES,), jnp.float32) for _ in range(D))
        inv_l = jnp.float32(1.0 / L)

        out_cp = [None] * n_groups
        for g in range(n_groups):
            buf = g % 2
            row0 = base_row + g * C
            if g + 1 < n_groups:
                idx_cp[g + 1] = pltpu.async_copy(
                    idx_hbm.at[pl.ds(row0 + C, C), :],
                    idx_v.at[1 - buf],
                    isems[1 - buf],
                )
            idx_cp[g].wait()
            if g >= 2:
                out_cp[g - 2].wait()

            def sub_body(s, carry2):
                rvec = s * _LANES + row_iota

                def l_body(l, accs):
                    lvec = jnp.full((_LANES,), l, jnp.int32)
                    vidx = plsc.load_gather(idx_v.at[buf], [rvec, lvec])
                    tbase = vidx * D
                    return tuple(
                        accs[d] + plsc.load_gather(table_v, [tbase + d])
                        for d in range(D)
                    )

                accs = lax.fori_loop(0, L, l_body, zeros, unroll=2)
                for d in range(D):
                    dvec = jnp.full((_LANES,), d, jnp.int32)
                    plsc.store_scatter(out_v.at[buf], [rvec, dvec], accs[d] * inv_l)
                return carry2

            lax.fori_loop(0, n_sub, sub_body, 0)
            out_cp[g] = pltpu.async_copy(
                out_v.at[buf], out_hbm.at[pl.ds(row0, C), :], osems[buf]
            )
        for g in range(max(0, n_groups - 2), n_groups):
            out_cp[g].wait()

    return pl.kernel(
        body,
        out_type=jax.ShapeDtypeStruct((B, D), jnp.float32),
        mesh=mesh,
        compiler_params=pltpu.CompilerParams(needs_layout_passes=False),
        scratch_types=[
            pltpu.VMEM((V * D,), jnp.float32),
            pltpu.VMEM((2, C, L), jnp.int32),
            pltpu.VMEM((2, C, D), jnp.float32),
            pltpu.SemaphoreType.DMA,
            pltpu.SemaphoreType.DMA,
            pltpu.SemaphoreType.DMA,
            pltpu.SemaphoreType.DMA,
        ],
    )(indices, emb_table.reshape(V * D))


def _mlp_body(x_ref, w1_ref, b1_ref, w2_ref, b2_ref, w3_ref, b3_ref, o_ref):
    x = x_ref[...]
    h1 = jnp.maximum(
        jnp.dot(x, w1_ref[...], preferred_element_type=jnp.float32) + b1_ref[...], 0.0
    )
    h2 = jnp.maximum(
        jnp.dot(h1, w2_ref[...], preferred_element_type=jnp.float32) + b2_ref[...], 0.0
    )
    o_ref[...] = jax.nn.sigmoid(
        jnp.dot(h2, w3_ref[...], preferred_element_type=jnp.float32) + b3_ref[...]
    )


def _mlp_tc(pooled, W1, b1, W2, b2, W3, b3):
    B, D = pooled.shape
    H1 = W1.shape[1]
    H2 = W2.shape[1]
    TB = 2048
    grid = (B // TB,)
    b1r, b2r, b3r = b1.reshape(1, H1), b2.reshape(1, H2), b3.reshape(1, 1)
    fixed = lambda i: (0, 0)
    return pl.pallas_call(
        _mlp_body,
        grid=grid,
        in_specs=[
            pl.BlockSpec((TB, D), lambda i: (i, 0)),
            pl.BlockSpec((D, H1), fixed),
            pl.BlockSpec((1, H1), fixed),
            pl.BlockSpec((H1, H2), fixed),
            pl.BlockSpec((1, H2), fixed),
            pl.BlockSpec((H2, 1), fixed),
            pl.BlockSpec((1, 1), fixed),
        ],
        out_specs=pl.BlockSpec((TB, 1), lambda i: (i, 0)),
        out_shape=jax.ShapeDtypeStruct((B, 1), jnp.float32),
    )(pooled, W1, b1r, W2, b2r, W3, b3r)


def kernel(indices, emb_table, W1, b1, W2, b2, W3, b3):
    pooled = _pooled_sc_kernel(indices, emb_table)
    return _mlp_tc(pooled, W1, b1, W2, b2, W3, b3)


# speedup vs baseline: 1.4538x; 1.0064x over previous
import functools

import jax
import jax.numpy as jnp
from jax import lax
from jax.experimental import pallas as pl
from jax.experimental.pallas import tpu as pltpu
from jax.experimental.pallas import tpu_sc as plsc

_LANES = 16


def _pooled_sc_kernel(indices, emb_table):
    B, L = indices.shape
    V, D = emb_table.shape

    info = plsc.get_sparse_core_info()
    NC, NS = info.num_cores, info.num_subcores
    NW = NC * NS

    rows_per_w = B // NW
    C = 64
    n_groups = rows_per_w // C
    n_sub = C // _LANES

    mesh = plsc.VectorSubcoreMesh(core_axis_name="c", subcore_axis_name="s")

    def body(idx_hbm, table_hbm, out_hbm, table_v, idx_v, out_v,
             isem0, isem1, osem0, osem1):
        wid = lax.axis_index("s") * NC + lax.axis_index("c")
        base_row = wid * rows_per_w
        isems = (isem0, isem1)
        osems = (osem0, osem1)

        idx_cp = [None] * n_groups
        idx_cp[0] = pltpu.async_copy(
            idx_hbm.at[pl.ds(base_row, C), :], idx_v.at[0], isems[0]
        )
        pltpu.sync_copy(table_hbm, table_v)

        row_iota = lax.iota(jnp.int32, _LANES)
        zeros = tuple(jnp.zeros((_LANES,), jnp.float32) for _ in range(D))
        inv_l = jnp.float32(1.0 / L)

        out_cp = [None] * n_groups
        for g in range(n_groups):
            buf = g % 2
            row0 = base_row + g * C
            if g + 1 < n_groups:
                idx_cp[g + 1] = pltpu.async_copy(
                    idx_hbm.at[pl.ds(row0 + C, C), :],
                    idx_v.at[1 - buf],
                    isems[1 - buf],
                )
            idx_cp[g].wait()
            if g >= 2:
                out_cp[g - 2].wait()

            def sub_body(s, carry2):
                rvec = s * _LANES + row_iota

                def l_body(l, accs):
                    lvec = jnp.full((_LANES,), l, jnp.int32)
                    vidx = plsc.load_gather(idx_v.at[buf], [rvec, lvec])
                    tbase = vidx * D
                    return tuple(
                        accs[d] + plsc.load_gather(table_v, [tbase + d])
                        for d in range(D)
                    )

                accs = lax.fori_loop(0, L, l_body, zeros, unroll=2)
                for d in range(D):
                    dvec = jnp.full((_LANES,), d, jnp.int32)
                    plsc.store_scatter(out_v.at[buf], [rvec, dvec], accs[d] * inv_l)
                return carry2

            lax.fori_loop(0, n_sub, sub_body, 0)
            out_cp[g] = pltpu.async_copy(
                out_v.at[buf], out_hbm.at[pl.ds(row0, C), :], osems[buf]
            )
        for g in range(max(0, n_groups - 2), n_groups):
            out_cp[g].wait()

    return pl.kernel(
        body,
        out_type=jax.ShapeDtypeStruct((B, D), jnp.float32),
        mesh=mesh,
        compiler_params=pltpu.CompilerParams(needs_layout_passes=False),
        scratch_types=[
            pltpu.VMEM((V * D,), jnp.float32),
            pltpu.VMEM((2, C, L), jnp.int32),
            pltpu.VMEM((2, C, D), jnp.float32),
            pltpu.SemaphoreType.DMA,
            pltpu.SemaphoreType.DMA,
            pltpu.SemaphoreType.DMA,
            pltpu.SemaphoreType.DMA,
        ],
    )(indices, emb_table.reshape(V * D))


def _mlp_body(x_ref, w1_ref, b1_ref, w2_ref, b2_ref, w3_ref, b3_ref, o_ref):
    x = x_ref[...]
    h1 = jnp.maximum(
        jnp.dot(x, w1_ref[...], preferred_element_type=jnp.float32) + b1_ref[...], 0.0
    )
    h2 = jnp.maximum(
        jnp.dot(h1, w2_ref[...], preferred_element_type=jnp.float32) + b2_ref[...], 0.0
    )
    o_ref[...] = jax.nn.sigmoid(
        jnp.dot(h2, w3_ref[...], preferred_element_type=jnp.float32) + b3_ref[...]
    )


def _mlp_tc(pooled, W1, b1, W2, b2, W3, b3):
    B, D = pooled.shape
    H1 = W1.shape[1]
    H2 = W2.shape[1]
    TB = B
    grid = (B // TB,)
    b1r, b2r, b3r = b1.reshape(1, H1), b2.reshape(1, H2), b3.reshape(1, 1)
    fixed = lambda i: (0, 0)
    return pl.pallas_call(
        _mlp_body,
        grid=grid,
        in_specs=[
            pl.BlockSpec((TB, D), lambda i: (i, 0)),
            pl.BlockSpec((D, H1), fixed),
            pl.BlockSpec((1, H1), fixed),
            pl.BlockSpec((H1, H2), fixed),
            pl.BlockSpec((1, H2), fixed),
            pl.BlockSpec((H2, 1), fixed),
            pl.BlockSpec((1, 1), fixed),
        ],
        out_specs=pl.BlockSpec((TB, 1), lambda i: (i, 0)),
        out_shape=jax.ShapeDtypeStruct((B, 1), jnp.float32),
    )(pooled, W1, b1r, W2, b2r, W3, b3r)


def kernel(indices, emb_table, W1, b1, W2, b2, W3, b3):
    pooled = _pooled_sc_kernel(indices, emb_table)
    return _mlp_tc(pooled, W1, b1, W2, b2, W3, b3)
